# baseline (speedup 1.0000x reference)
import functools

import jax
import jax.numpy as jnp
from jax import lax
from jax.experimental import pallas as pl
from jax.experimental.pallas import tpu as pltpu
from jax.experimental.pallas import tpu_sc as plsc

NC, NS = 2, 16
NW = NC * NS
CHUNK = 128
EPS = 1e-5


def _sc_gather(i0, i1, i2, t0, t1, t2, n_tokens, dp):
    per_w = n_tokens // NW
    chunks = per_w // CHUNK
    dt = t0.dtype
    mesh = plsc.VectorSubcoreMesh(
        core_axis_name="c", subcore_axis_name="s", num_cores=NC,
        num_subcores=NS)
    out_t = [jax.ShapeDtypeStruct((n_tokens, dp), dt)] * 3
    scratch = (
        [pltpu.VMEM((chunks, CHUNK), jnp.int32) for _ in range(3)]
        + [pltpu.VMEM((CHUNK, dp), dt) for _ in range(6)]
        + [pltpu.SemaphoreType.DMA] * 4
    )

    @functools.partial(pl.kernel, out_type=out_t, mesh=mesh,
                       scratch_types=scratch,
                       compiler_params=pltpu.CompilerParams(
                           use_tc_tiling_on_sc=False))
    def k(ih0, ih1, ih2, th0, th1, th2, o0, o1, o2,
          iv0, iv1, iv2, ra0, ra1, ra2, rb0, rb1, rb2,
          g0, g1, w0, w1):
        wid = lax.axis_index("s") * NC + lax.axis_index("c")
        base = wid * per_w
        ivs = (iv0, iv1, iv2)
        ths = (th0, th1, th2)
        outs = (o0, o1, o2)
        bufs = ((ra0, ra1, ra2), (rb0, rb1, rb2))
        gsem = (g0, g1)
        wsem = (w0, w1)
        for ih, iv in zip((ih0, ih1, ih2), ivs):
            pltpu.sync_copy(ih.at[wid], iv)

        def gathers(kc, s):
            return [pltpu.make_async_copy(th.at[iv.at[kc]], rv, gsem[s])
                    for th, iv, rv in zip(ths, ivs, bufs[s])]

        def writes(kc, s):
            return [pltpu.make_async_copy(
                        rv, ov.at[pl.ds(base + kc * CHUNK, CHUNK)], wsem[s])
                    for rv, ov in zip(bufs[s], outs)]

        for dsc in gathers(0, 0):
            dsc.start()

        def step(i, b, fire_next):
            kc = 2 * i + b
            for dsc in gathers(kc, b):
                dsc.wait()
            drain_prev = writes(kc, 1 - b)
            if b == 1:
                for dsc in drain_prev:
                    dsc.wait()
            else:
                @pl.when(i >= 1)
                def _():
                    for dsc in drain_prev:
                        dsc.wait()
            if fire_next:
                for dsc in gathers(kc + 1, 1 - b):
                    dsc.start()
            for dsc in writes(kc, b):
                dsc.start()

        def body(i, carry):
            step(i, 0, True)
            step(i, 1, True)
            return carry

        n_main = (chunks - 1) // 2
        lax.fori_loop(0, n_main, body, 0)
        if chunks % 2 == 1:
            step(n_main, 0, False)
            for dsc in writes(chunks - 1, 0):
                dsc.wait()
        else:
            step(n_main, 0, True)
            step(n_main, 1, False)
            for dsc in writes(chunks - 1, 1):
                dsc.wait()

    return k(i0, i1, i2, t0, t1, t2)


def _tc_dense(e0, e1, e2, interp, elp, w3, wTp, bc, wc, bct, g1, be1,
              g2, be2, n_tokens, hd, tile):
    grid = (n_tokens // tile,)
    intd = w3.shape[1]
    dp = e0.shape[1]
    rows = tile // 128

    def bcast(sp):
        t_r = lax.broadcasted_iota(jnp.int32, (tile, rows), 0) // 128
        k_r = lax.broadcasted_iota(jnp.int32, (tile, rows), 1)
        U = (t_r == k_r).astype(jnp.float32)
        G = lax.dot_general(U, sp, (((1,), (0,)), ((), ())),
                            preferred_element_type=jnp.float32)
        t_c = lax.broadcasted_iota(jnp.int32, (tile, 128), 0) % 128
        c_c = lax.broadcasted_iota(jnp.int32, (tile, 128), 1)
        sel = jnp.where(t_c == c_c, G, 0.0)
        ones = jnp.ones((128, 128), jnp.float32)
        return lax.dot_general(sel, ones, (((1,), (0,)), ((), ())),
                               preferred_element_type=jnp.float32)

    def body(e0r, e1r, e2r, intr, elr, w3r, wTr, bcr, wcr, bctr,
             g1r, be1r, g2r, be2r, out):
        dot = lambda a, b: lax.dot_general(
            a, b, (((1,), (0,)), ((), ())),
            preferred_element_type=jnp.float32)
        X = dot(e0r[...].astype(jnp.float32), wTr[0:dp, :])
        X = X + dot(e1r[...].astype(jnp.float32), wTr[dp:2 * dp, :])
        X = X + dot(e2r[...].astype(jnp.float32), wTr[2 * dp:3 * dp, :])
        w3p = dot(w3r[...], wTr[3 * dp:3 * dp + intd, :])
        ib = bcast(intr[...].astype(jnp.float32))
        arow = w3p[0:1, :]
        brow = w3p[1:2, :] - w3p[0:1, :]
        crow = w3p[2:3, :] - 2.0 * w3p[1:2, :] + w3p[0:1, :]
        X = X + arow + ib * brow + (ib * (ib - 1.0) * 0.5) * crow
        X = X + bcr[...]
        mu = jnp.mean(X, axis=-1, keepdims=True)
        var = jnp.mean((X - mu) ** 2, axis=-1, keepdims=True)
        Xn = (X - mu) * lax.rsqrt(var + EPS) * g1r[...] + be1r[...]
        eb = bcast(elr[...])
        Y = eb * wcr[...] + bctr[...]
        muY = jnp.mean(Y, axis=-1, keepdims=True)
        varY = jnp.mean((Y - muY) ** 2, axis=-1, keepdims=True)
        Yn = (Y - muY) * lax.rsqrt(varY + EPS) * g2r[...] + be2r[...]
        out[:, :hd] = Xn
        out[:, hd:] = Yn

    tok_spec = lambda w: pl.BlockSpec((tile, w), lambda i: (i, 0))
    pk_spec = pl.BlockSpec((rows, 128), lambda i: (i, 0))
    fix_spec = lambda s: pl.BlockSpec(s, lambda i: (0, 0))
    return pl.pallas_call(
        body,
        grid=grid,
        in_specs=[tok_spec(dp)] * 3 + [pk_spec, pk_spec,
                  fix_spec((3, intd)),
                  fix_spec((3 * dp + intd, hd))] + [fix_spec((1, hd))] * 7,
        out_specs=pl.BlockSpec((tile, 2 * hd), lambda i: (i, 0)),
        out_shape=jax.ShapeDtypeStruct((n_tokens, 2 * hd), jnp.float32),
        compiler_params=pltpu.CompilerParams(
            dimension_semantics=("arbitrary",)),
    )(e0, e1, e2, interp, elp, w3, wTp, bc, wc, bct, g1, be1, g2, be2)


def kernel(assessmentItemID, testId, KnowledgeTag, Interaction, elapsed,
           emb_item, emb_test, emb_tag, emb_inter,
           W_comb, b_comb, W_cont, b_cont,
           g_cat, beta_cat, g_cont, beta_cont):
    B, L = assessmentItemID.shape
    n = B * L
    intd = emb_item.shape[1]
    dp = 128
    hd = W_comb.shape[0]
    pad = lambda t: jnp.pad(t.astype(jnp.bfloat16), ((0, 0), (0, dp - intd)))
    slab = lambda a: a.reshape(NW, n // (NW * CHUNK), CHUNK)
    e0, e1, e2 = _sc_gather(
        slab(assessmentItemID), slab(testId), slab(KnowledgeTag),
        pad(emb_item), pad(emb_test), pad(emb_tag), n, dp)
    wT = W_comb.T
    z = jnp.zeros((dp - intd, hd), jnp.float32)
    wTp = jnp.concatenate(
        [wT[0:intd], z, wT[intd:2 * intd], z, wT[2 * intd:3 * intd], z,
         wT[3 * intd:4 * intd]], axis=0)
    row = lambda v: v.reshape(1, hd)
    out = _tc_dense(
        e0, e1, e2, Interaction.reshape(n // 128, 128),
        elapsed.reshape(n // 128, 128), emb_inter, wTp, row(b_comb),
        W_cont.reshape(1, hd), row(b_cont), row(g_cat), row(beta_cat),
        row(g_cont), row(beta_cont), n, hd, tile=2048)
    return out.reshape(B, L, 2 * hd), B


# speedup vs baseline: 2.2011x; 2.2011x over previous
import functools

import jax
import jax.numpy as jnp
from jax import lax
from jax.experimental import pallas as pl
from jax.experimental.pallas import tpu as pltpu
from jax.experimental.pallas import tpu_sc as plsc

NC, NS = 2, 16
NW = NC * NS
CHUNK = 128
EPS = 1e-5


def _sc_gather(i0, i1, i2, t0, t1, t2, n_tokens, dp):
    per_w = n_tokens // NW
    chunks = per_w // CHUNK
    dt = t0.dtype
    mesh = plsc.VectorSubcoreMesh(
        core_axis_name="c", subcore_axis_name="s", num_cores=NC,
        num_subcores=NS)
    out_t = [jax.ShapeDtypeStruct((n_tokens, dp), dt)] * 3
    scratch = (
        [pltpu.VMEM((chunks, CHUNK), jnp.int32) for _ in range(3)]
        + [pltpu.VMEM((CHUNK, dp), dt) for _ in range(6)]
        + [pltpu.SemaphoreType.DMA] * 4
    )

    @functools.partial(pl.kernel, out_type=out_t, mesh=mesh,
                       scratch_types=scratch,
                       compiler_params=pltpu.CompilerParams(
                           use_tc_tiling_on_sc=False))
    def k(ih0, ih1, ih2, th0, th1, th2, o0, o1, o2,
          iv0, iv1, iv2, ra0, ra1, ra2, rb0, rb1, rb2,
          g0, g1, w0, w1):
        wid = lax.axis_index("s") * NC + lax.axis_index("c")
        base = wid * per_w
        ivs = (iv0, iv1, iv2)
        ths = (th0, th1, th2)
        outs = (o0, o1, o2)
        bufs = ((ra0, ra1, ra2), (rb0, rb1, rb2))
        gsem = (g0, g1)
        wsem = (w0, w1)
        for ih, iv in zip((ih0, ih1, ih2), ivs):
            pltpu.sync_copy(ih.at[wid], iv)

        def gathers(kc, s):
            return [pltpu.make_async_copy(th.at[iv.at[kc]], rv, gsem[s])
                    for th, iv, rv in zip(ths, ivs, bufs[s])]

        def writes(kc, s):
            return [pltpu.make_async_copy(
                        rv, ov.at[pl.ds(base + kc * CHUNK, CHUNK)], wsem[s])
                    for rv, ov in zip(bufs[s], outs)]

        for dsc in gathers(0, 0):
            dsc.start()

        def step(i, b, fire_next):
            kc = 2 * i + b
            for dsc in gathers(kc, b):
                dsc.wait()
            drain_prev = writes(kc, 1 - b)
            if b == 1:
                for dsc in drain_prev:
                    dsc.wait()
            else:
                @pl.when(i >= 1)
                def _():
                    for dsc in drain_prev:
                        dsc.wait()
            if fire_next:
                for dsc in gathers(kc + 1, 1 - b):
                    dsc.start()
            for dsc in writes(kc, b):
                dsc.start()

        def body(i, carry):
            step(i, 0, True)
            step(i, 1, True)
            return carry

        n_main = (chunks - 1) // 2
        lax.fori_loop(0, n_main, body, 0)
        if chunks % 2 == 1:
            step(n_main, 0, False)
            for dsc in writes(chunks - 1, 0):
                dsc.wait()
        else:
            step(n_main, 0, True)
            step(n_main, 1, False)
            for dsc in writes(chunks - 1, 1):
                dsc.wait()

    return k(i0, i1, i2, t0, t1, t2)


def _tc_dense(e0, e1, e2, interp, elp, w3, wTp, bc, wc, bct, g1, be1,
              g2, be2, n_tokens, hd, tile, goff=0, prev=None):
    grid = (e0.shape[0] // tile,)
    intd = w3.shape[1]
    dp = e0.shape[1]
    rows = tile // 128

    def bcast(sp):
        t_r = lax.broadcasted_iota(jnp.int32, (tile, rows), 0) // 128
        k_r = lax.broadcasted_iota(jnp.int32, (tile, rows), 1)
        U = (t_r == k_r).astype(jnp.float32)
        G = lax.dot_general(U, sp, (((1,), (0,)), ((), ())),
                            preferred_element_type=jnp.float32)
        t_c = lax.broadcasted_iota(jnp.int32, (tile, 128), 0) % 128
        c_c = lax.broadcasted_iota(jnp.int32, (tile, 128), 1)
        sel = jnp.where(t_c == c_c, G, 0.0)
        ones = jnp.ones((128, 128), jnp.float32)
        return lax.dot_general(sel, ones, (((1,), (0,)), ((), ())),
                               preferred_element_type=jnp.float32)

    def body(e0r, e1r, e2r, intr, elr, w3r, wTr, bcr, wcr, bctr,
             g1r, be1r, g2r, be2r, *rest):
        out = rest[-1]
        dot = lambda a, b: lax.dot_general(
            a, b, (((1,), (0,)), ((), ())),
            preferred_element_type=jnp.float32)
        X = dot(e0r[...].astype(jnp.float32), wTr[0:dp, :])
        X = X + dot(e1r[...].astype(jnp.float32), wTr[dp:2 * dp, :])
        X = X + dot(e2r[...].astype(jnp.float32), wTr[2 * dp:3 * dp, :])
        w3p = dot(w3r[...], wTr[3 * dp:3 * dp + intd, :])
        ib = bcast(intr[...].astype(jnp.float32))
        arow = w3p[0:1, :]
        brow = w3p[1:2, :] - w3p[0:1, :]
        crow = w3p[2:3, :] - 2.0 * w3p[1:2, :] + w3p[0:1, :]
        X = X + arow + ib * brow + (ib * (ib - 1.0) * 0.5) * crow
        X = X + bcr[...]
        mu = jnp.mean(X, axis=-1, keepdims=True)
        var = jnp.mean((X - mu) ** 2, axis=-1, keepdims=True)
        Xn = (X - mu) * lax.rsqrt(var + EPS) * g1r[...] + be1r[...]
        eb = bcast(elr[...])
        Y = eb * wcr[...] + bctr[...]
        muY = jnp.mean(Y, axis=-1, keepdims=True)
        varY = jnp.mean((Y - muY) ** 2, axis=-1, keepdims=True)
        Yn = (Y - muY) * lax.rsqrt(varY + EPS) * g2r[...] + be2r[...]
        out[:, :hd] = Xn
        out[:, hd:] = Yn

    tok_spec = lambda w: pl.BlockSpec((tile, w), lambda i: (i, 0))
    pk_spec = pl.BlockSpec((rows, 128), lambda i: (i + goff, 0))
    fix_spec = lambda s: pl.BlockSpec(s, lambda i: (0, 0))
    in_specs = [tok_spec(dp)] * 3 + [pk_spec, pk_spec,
                fix_spec((3, intd)),
                fix_spec((3 * dp + intd, hd))] + [fix_spec((1, hd))] * 7
    args = [e0, e1, e2, interp, elp, w3, wTp, bc, wc, bct, g1, be1,
            g2, be2]
    aliases = {}
    if prev is not None:
        in_specs.append(pl.BlockSpec(memory_space=pl.ANY))
        args.append(prev)
        aliases = {14: 0}
    return pl.pallas_call(
        body,
        grid=grid,
        in_specs=in_specs,
        out_specs=pl.BlockSpec((tile, 2 * hd), lambda i: (i + goff, 0)),
        out_shape=jax.ShapeDtypeStruct((n_tokens, 2 * hd), jnp.float32),
        input_output_aliases=aliases,
        compiler_params=pltpu.CompilerParams(
            dimension_semantics=("arbitrary",)),
    )(*args)


def kernel(assessmentItemID, testId, KnowledgeTag, Interaction, elapsed,
           emb_item, emb_test, emb_tag, emb_inter,
           W_comb, b_comb, W_cont, b_cont,
           g_cat, beta_cat, g_cont, beta_cont):
    B, L = assessmentItemID.shape
    n = B * L
    intd = emb_item.shape[1]
    dp = 128
    hd = W_comb.shape[0]
    pad = lambda t: jnp.pad(t, ((0, 0), (0, dp - intd)))
    t0, t1, t2 = pad(emb_item), pad(emb_test), pad(emb_tag)
    nh = n // 2
    halves = lambda a: a.reshape(2, NW, nh // (NW * CHUNK), CHUNK)
    i0h, i1h, i2h = (halves(assessmentItemID), halves(testId),
                     halves(KnowledgeTag))
    eA = _sc_gather(i0h[0], i1h[0], i2h[0], t0, t1, t2, nh, dp)
    eB = _sc_gather(i0h[1], i1h[1], i2h[1], t0, t1, t2, nh, dp)
    wT = W_comb.T
    z = jnp.zeros((dp - intd, hd), jnp.float32)
    wTp = jnp.concatenate(
        [wT[0:intd], z, wT[intd:2 * intd], z, wT[2 * intd:3 * intd], z,
         wT[3 * intd:4 * intd]], axis=0)
    row = lambda v: v.reshape(1, hd)
    interp = Interaction.reshape(n // 128, 128)
    elp = elapsed.reshape(n // 128, 128)
    tile = 2048
    params = (emb_inter, wTp, row(b_comb), W_cont.reshape(1, hd),
              row(b_cont), row(g_cat), row(beta_cat), row(g_cont),
              row(beta_cont))
    outA = _tc_dense(*eA, interp, elp, *params, n, hd, tile, goff=0)
    out = _tc_dense(*eB, interp, elp, *params, n, hd, tile,
                    goff=nh // tile, prev=outA)
    return out.reshape(B, L, 2 * hd), B


# speedup vs baseline: 2.2338x; 1.0149x over previous
import functools

import jax
import jax.numpy as jnp
from jax import lax
from jax.experimental import pallas as pl
from jax.experimental.pallas import tpu as pltpu
from jax.experimental.pallas import tpu_sc as plsc

NC, NS = 2, 16
NW = NC * NS
CHUNK = 100
NSPLIT = 4
EPS = 1e-5


def _sc_gather(i0, i1, i2, t0, t1, t2, n_tokens, dp):
    per_w = n_tokens // NW
    chunks = per_w // CHUNK
    dt = t0.dtype
    mesh = plsc.VectorSubcoreMesh(
        core_axis_name="c", subcore_axis_name="s", num_cores=NC,
        num_subcores=NS)
    out_t = [jax.ShapeDtypeStruct((n_tokens, dp), dt)] * 3
    scratch = (
        [pltpu.VMEM((chunks, CHUNK), jnp.int32) for _ in range(3)]
        + [pltpu.VMEM((CHUNK, dp), dt) for _ in range(6)]
        + [pltpu.SemaphoreType.DMA] * 4
    )

    @functools.partial(pl.kernel, out_type=out_t, mesh=mesh,
                       scratch_types=scratch,
                       compiler_params=pltpu.CompilerParams(
                           use_tc_tiling_on_sc=False))
    def k(ih0, ih1, ih2, th0, th1, th2, o0, o1, o2,
          iv0, iv1, iv2, ra0, ra1, ra2, rb0, rb1, rb2,
          g0, g1, w0, w1):
        wid = lax.axis_index("s") * NC + lax.axis_index("c")
        base = wid * per_w
        ivs = (iv0, iv1, iv2)
        ths = (th0, th1, th2)
        outs = (o0, o1, o2)
        bufs = ((ra0, ra1, ra2), (rb0, rb1, rb2))
        gsem = (g0, g1)
        wsem = (w0, w1)
        for ih, iv in zip((ih0, ih1, ih2), ivs):
            pltpu.sync_copy(ih.at[wid], iv)

        def gathers(kc, s):
            return [pltpu.make_async_copy(th.at[iv.at[kc]], rv, gsem[s])
                    for th, iv, rv in zip(ths, ivs, bufs[s])]

        def writes(kc, s):
            return [pltpu.make_async_copy(
                        rv, ov.at[pl.ds(base + kc * CHUNK, CHUNK)], wsem[s])
                    for rv, ov in zip(bufs[s], outs)]

        for dsc in gathers(0, 0):
            dsc.start()

        def step(i, b, fire_next):
            kc = 2 * i + b
            for dsc in gathers(kc, b):
                dsc.wait()
            drain_prev = writes(kc, 1 - b)
            if b == 1:
                for dsc in drain_prev:
                    dsc.wait()
            else:
                @pl.when(i >= 1)
                def _():
                    for dsc in drain_prev:
                        dsc.wait()
            if fire_next:
                for dsc in gathers(kc + 1, 1 - b):
                    dsc.start()
            for dsc in writes(kc, b):
                dsc.start()

        def body(i, carry):
            step(i, 0, True)
            step(i, 1, True)
            return carry

        n_main = (chunks - 1) // 2
        lax.fori_loop(0, n_main, body, 0)
        if chunks % 2 == 1:
            step(n_main, 0, False)
            for dsc in writes(chunks - 1, 0):
                dsc.wait()
        else:
            step(n_main, 0, True)
            step(n_main, 1, False)
            for dsc in writes(chunks - 1, 1):
                dsc.wait()

    return k(i0, i1, i2, t0, t1, t2)


def _tc_dense(e0, e1, e2, interp, elp, w3, wTp, bc, wc, bct, g1, be1,
              g2, be2, n_tokens, hd, tile, goff=0, prev=None):
    grid = (e0.shape[0] // tile,)
    intd = w3.shape[1]
    dp = e0.shape[1]
    rows = tile // 128

    def bcast(sp):
        t_r = lax.broadcasted_iota(jnp.int32, (tile, rows), 0) // 128
        k_r = lax.broadcasted_iota(jnp.int32, (tile, rows), 1)
        U = (t_r == k_r).astype(jnp.float32)
        G = lax.dot_general(U, sp, (((1,), (0,)), ((), ())),
                            preferred_element_type=jnp.float32)
        t_c = lax.broadcasted_iota(jnp.int32, (tile, 128), 0) % 128
        c_c = lax.broadcasted_iota(jnp.int32, (tile, 128), 1)
        sel = jnp.where(t_c == c_c, G, 0.0)
        ones = jnp.ones((128, 128), jnp.float32)
        return lax.dot_general(sel, ones, (((1,), (0,)), ((), ())),
                               preferred_element_type=jnp.float32)

    def body(e0r, e1r, e2r, intr, elr, w3r, wTr, bcr, wcr, bctr,
             g1r, be1r, g2r, be2r, *rest):
        out = rest[-1]
        dot = lambda a, b: lax.dot_general(
            a, b, (((1,), (0,)), ((), ())),
            preferred_element_type=jnp.float32)
        X = dot(e0r[...].astype(jnp.float32), wTr[0:dp, :])
        X = X + dot(e1r[...].astype(jnp.float32), wTr[dp:2 * dp, :])
        X = X + dot(e2r[...].astype(jnp.float32), wTr[2 * dp:3 * dp, :])
        w3p = dot(w3r[...], wTr[3 * dp:3 * dp + intd, :])
        ib = bcast(intr[...].astype(jnp.float32))
        arow = w3p[0:1, :]
        brow = w3p[1:2, :] - w3p[0:1, :]
        crow = w3p[2:3, :] - 2.0 * w3p[1:2, :] + w3p[0:1, :]
        X = X + arow + ib * brow + (ib * (ib - 1.0) * 0.5) * crow
        X = X + bcr[...]
        mu = jnp.mean(X, axis=-1, keepdims=True)
        var = jnp.mean((X - mu) ** 2, axis=-1, keepdims=True)
        Xn = (X - mu) * lax.rsqrt(var + EPS) * g1r[...] + be1r[...]
        eb = bcast(elr[...])
        Y = eb * wcr[...] + bctr[...]
        muY = jnp.mean(Y, axis=-1, keepdims=True)
        varY = jnp.mean((Y - muY) ** 2, axis=-1, keepdims=True)
        Yn = (Y - muY) * lax.rsqrt(varY + EPS) * g2r[...] + be2r[...]
        out[:, :hd] = Xn
        out[:, hd:] = Yn

    tok_spec = lambda w: pl.BlockSpec((tile, w), lambda i: (i, 0))
    pk_spec = pl.BlockSpec((rows, 128), lambda i: (i + goff, 0))
    fix_spec = lambda s: pl.BlockSpec(s, lambda i: (0, 0))
    in_specs = [tok_spec(dp)] * 3 + [pk_spec, pk_spec,
                fix_spec((3, intd)),
                fix_spec((3 * dp + intd, hd))] + [fix_spec((1, hd))] * 7
    args = [e0, e1, e2, interp, elp, w3, wTp, bc, wc, bct, g1, be1,
            g2, be2]
    aliases = {}
    if prev is not None:
        in_specs.append(pl.BlockSpec(memory_space=pl.ANY))
        args.append(prev)
        aliases = {14: 0}
    return pl.pallas_call(
        body,
        grid=grid,
        in_specs=in_specs,
        out_specs=pl.BlockSpec((tile, 2 * hd), lambda i: (i + goff, 0)),
        out_shape=jax.ShapeDtypeStruct((n_tokens, 2 * hd), jnp.float32),
        input_output_aliases=aliases,
        compiler_params=pltpu.CompilerParams(
            dimension_semantics=("arbitrary",)),
    )(*args)


def kernel(assessmentItemID, testId, KnowledgeTag, Interaction, elapsed,
           emb_item, emb_test, emb_tag, emb_inter,
           W_comb, b_comb, W_cont, b_cont,
           g_cat, beta_cat, g_cont, beta_cont):
    B, L = assessmentItemID.shape
    n = B * L
    intd = emb_item.shape[1]
    dp = 128
    hd = W_comb.shape[0]
    pad = lambda t: jnp.pad(t, ((0, 0), (0, dp - intd)))
    t0, t1, t2 = pad(emb_item), pad(emb_test), pad(emb_tag)
    nh = n // NSPLIT
    parts = lambda a: a.reshape(NSPLIT, NW, nh // (NW * CHUNK), CHUNK)
    i0h, i1h, i2h = (parts(assessmentItemID), parts(testId),
                     parts(KnowledgeTag))
    es = [_sc_gather(i0h[q], i1h[q], i2h[q], t0, t1, t2, nh, dp)
          for q in range(NSPLIT)]
    wT = W_comb.T
    z = jnp.zeros((dp - intd, hd), jnp.float32)
    wTp = jnp.concatenate(
        [wT[0:intd], z, wT[intd:2 * intd], z, wT[2 * intd:3 * intd], z,
         wT[3 * intd:4 * intd]], axis=0)
    row = lambda v: v.reshape(1, hd)
    interp = Interaction.reshape(n // 128, 128)
    elp = elapsed.reshape(n // 128, 128)
    tile = 2048
    params = (emb_inter, wTp, row(b_comb), W_cont.reshape(1, hd),
              row(b_cont), row(g_cat), row(beta_cat), row(g_cont),
              row(beta_cont))
    out = None
    for q in range(NSPLIT):
        out = _tc_dense(*es[q], interp, elp, *params, n, hd, tile,
                        goff=q * (nh // tile), prev=out)
    return out.reshape(B, L, 2 * hd), B
